# Initial kernel scaffold; baseline (speedup 1.0000x reference)
import jax
import jax.numpy as jnp
from jax.experimental import pallas as pl


def kernel(inputs, embeddings):
    raise NotImplementedError("write your pallas kernel here")


# speedup vs baseline: 1.4785x; 1.4785x over previous
import functools

import jax
import jax.numpy as jnp
from jax import lax
from jax.experimental import pallas as pl
from jax.experimental.pallas import tpu as pltpu
from jax.experimental.pallas import tpu_sc as plsc

_VOCAB = 1000000
_EMBED = 32
_BATCH = 4096
_HIST = 200
_TOTAL = _BATCH * _HIST

_NC = 2
_NS = 16
_NW = _NC * _NS
_PER_W = _TOTAL // _NW
_CHUNK = 1600
_NCHUNKS = _PER_W // _CHUNK


def _sc_gather(idx_flat, table):
  mesh = plsc.VectorSubcoreMesh(core_axis_name="c", subcore_axis_name="s")

  @functools.partial(
      pl.kernel,
      mesh=mesh,
      out_type=jax.ShapeDtypeStruct((_TOTAL, _EMBED), jnp.float32),
      scratch_types=[
          pltpu.VMEM((_CHUNK,), jnp.int32),
          pltpu.VMEM((_CHUNK, _EMBED), jnp.float32),
          pltpu.SemaphoreType.DMA,
      ],
      compiler_params=pltpu.CompilerParams(use_tc_tiling_on_sc=False),
  )
  def k(idx_hbm, table_hbm, out_hbm, idx_v, rows_v, sem):
    wid = lax.axis_index("s") * _NC + lax.axis_index("c")
    base = wid * _PER_W

    def body(i, _):
      off = base + i * _CHUNK
      pltpu.sync_copy(idx_hbm.at[pl.ds(off, _CHUNK)], idx_v)
      pltpu.async_copy(table_hbm.at[idx_v], rows_v, sem).wait()
      pltpu.sync_copy(rows_v, out_hbm.at[pl.ds(off, _CHUNK)])
      return 0

    lax.fori_loop(0, _NCHUNKS, body, 0)

  return k(idx_flat, table)


def kernel(inputs, embeddings):
  idx_flat = inputs.reshape(-1).astype(jnp.int32)
  out = _sc_gather(idx_flat, embeddings)
  return out.reshape(_BATCH, _HIST, _EMBED)


# speedup vs baseline: 1.4985x; 1.0135x over previous
import functools

import jax
import jax.numpy as jnp
from jax import lax
from jax.experimental import pallas as pl
from jax.experimental.pallas import tpu as pltpu
from jax.experimental.pallas import tpu_sc as plsc

_VOCAB = 1000000
_EMBED = 32
_BATCH = 4096
_HIST = 200
_TOTAL = _BATCH * _HIST

_NC = 2
_NS = 16
_NW = _NC * _NS
_PER_W = _TOTAL // _NW
_REGION = 1600
_NSTREAM = 8
_SUB = _REGION // _NSTREAM
_NREGIONS = _PER_W // _REGION


def _sc_gather(idx_flat, table):
  mesh = plsc.VectorSubcoreMesh(core_axis_name="c", subcore_axis_name="s")

  @functools.partial(
      pl.kernel,
      mesh=mesh,
      out_type=jax.ShapeDtypeStruct((_TOTAL, _EMBED), jnp.float32),
      scratch_types=[
          pltpu.VMEM((_PER_W,), jnp.int32),
          pltpu.VMEM((2, _REGION, _EMBED), jnp.float32),
          pltpu.SemaphoreType.DMA,
          pltpu.SemaphoreType.DMA,
          pltpu.SemaphoreType.DMA,
          pltpu.SemaphoreType.DMA,
      ],
      compiler_params=pltpu.CompilerParams(use_tc_tiling_on_sc=False),
  )
  def k(idx_hbm, table_hbm, out_hbm, idx_v, rows_v, g0, g1, s0, s1):
    wid = lax.axis_index("s") * _NC + lax.axis_index("c")
    base = wid * _PER_W
    gsem = (g0, g1)
    ssem = (s0, s1)

    pltpu.sync_copy(idx_hbm.at[pl.ds(base, _PER_W)], idx_v)

    def gathers_start(i, b):
      descs = []
      for j in range(_NSTREAM):
        off = i * _REGION + j * _SUB
        descs.append(pltpu.async_copy(
            table_hbm.at[idx_v.at[pl.ds(off, _SUB)]],
            rows_v.at[b, pl.ds(j * _SUB, _SUB)], gsem[b]))
      return descs

    def store_start(i, b):
      return pltpu.async_copy(
          rows_v.at[b], out_hbm.at[pl.ds(base + i * _REGION, _REGION)],
          ssem[b])

    gathers = [None, None]
    stores = [None, None]
    gathers[0] = gathers_start(0, 0)
    for i in range(_NREGIONS):
      b = i % 2
      if i + 1 < _NREGIONS:
        if stores[1 - b] is not None:
          stores[1 - b].wait()
        gathers[1 - b] = gathers_start(i + 1, 1 - b)
      for d in gathers[b]:
        d.wait()
      stores[b] = store_start(i, b)
    stores[0].wait()
    stores[1].wait()

  return k(idx_flat, table)


def kernel(inputs, embeddings):
  idx_flat = inputs.reshape(-1).astype(jnp.int32)
  out = _sc_gather(idx_flat, embeddings)
  return out.reshape(_BATCH, _HIST, _EMBED)
